# Initial kernel scaffold; baseline (speedup 1.0000x reference)
import jax
import jax.numpy as jnp
from jax.experimental import pallas as pl


def kernel(x, mids, gate_W, moe0_W1, moe0_b1, moe0_W2, moe0_b2, atom1_W, atom1_b, atom2_W, atom2_b):
    raise NotImplementedError("write your pallas kernel here")


# speedup vs baseline: 2.1048x; 2.1048x over previous
import functools

import jax
import jax.numpy as jnp
from jax.experimental import pallas as pl

B = 32
NCLS = 6
P = 196
D = 768
H = 3072
OUT = 768

_PAIRS = [
    (3, 0, 0, 1), (4, 0, 1, 1),
    (3, 1, 2, 1), (4, 1, 3, 1),
    (3, 2, 4, 1), (4, 2, 5, 1),
    (0, 3, 0, 0), (1, 3, 2, 0), (2, 3, 4, 0),
    (0, 4, 1, 0), (1, 4, 3, 0), (2, 4, 5, 0),
]
_PA = tuple(p[0] for p in _PAIRS)
_PB = tuple(p[1] for p in _PAIRS)
_PI = tuple(p[2] for p in _PAIRS)
_PJ = tuple(p[3] for p in _PAIRS)
_TOKEN_PAIRS = [(0, 6), (1, 9), (2, 7), (3, 10), (4, 8), (5, 11)]

_MT = 3136
_HT = 512


def _bf(v):
    return v.astype(jnp.bfloat16)


def _pi(p):
    return jnp.where(p < 6, p, jnp.where(p < 9, 2 * (p - 6), 2 * (p - 9) + 1))


def _pa(p):
    return jnp.where(p < 6, 3 + p % 2, (p - 6) % 3)


def _pb(p):
    return jnp.where(p < 6, p // 2, jnp.where(p < 9, 3, 4))


def _gelu(v):
    return 0.5 * v * (1.0 + jax.lax.erf(v * 0.7071067811865476))


def _dot_t(a, b):
    return jax.lax.dot_general(
        _bf(a), _bf(b), (((1,), (1,)), ((), ())),
        preferred_element_type=jnp.float32)


def _patch_body(x_ref, w1_ref, b1_ref, w2_ref, b2_ref, o_ref):
    h = pl.program_id(1)
    z = _dot_t(x_ref[...], w1_ref[...]) + b1_ref[...]
    z = _gelu(z)
    contrib = _dot_t(z, w2_ref[...])

    @pl.when(h == 0)
    def _():
        o_ref[...] = contrib + b2_ref[...]

    @pl.when(h != 0)
    def _():
        o_ref[...] += contrib


def _stage1_body(t_ref, w_ref, b_ref, h_ref):
    z = _dot_t(t_ref[...], w_ref[0]) + b_ref[0]
    h_ref[0] = _bf(_gelu(z))


def _stage2_body(tok_ref, gw_ref, h_ref, w_ref, b_ref, o_ref):
    o = jax.lax.dot_general(
        h_ref[0], _bf(w_ref[0]), (((1,), (1,)), ((), ())),
        preferred_element_type=jnp.float32) + b_ref[0]
    g = _dot_t(tok_ref[0], gw_ref[0])
    m = jnp.max(g, axis=-1, keepdims=True)
    e = jnp.exp(g - m)
    s = e / jnp.sum(e, axis=-1, keepdims=True)
    gk = jnp.min(s, axis=-1, keepdims=True)
    w = (s - gk > 0).astype(jnp.float32)
    p = pl.program_id(0)
    wj = jnp.where(p < 6, w[:, 1:2], w[:, 0:1])
    o_ref[0] = o * wj


def _combine_body(p_ref, o_ref):
    for i, (p0, p1) in enumerate(_TOKEN_PAIRS):
        o_ref[i] = p_ref[p0] + p_ref[p1]


def kernel(x, mids, gate_W, moe0_W1, moe0_b1, moe0_W2, moe0_b2,
           atom1_W, atom1_b, atom2_W, atom2_b):
    del mids
    patch = x[:, NCLS:, :].reshape(B * P, D)
    toks = x[:, :NCLS, :].transpose(1, 0, 2)
    tflat = toks.reshape(NCLS * B, D)
    b1r = moe0_b1.reshape(1, H)
    b2r = moe0_b2.reshape(1, OUT)
    a1b = atom1_b.reshape(5, 1, H)
    a2b = atom2_b.reshape(5, 1, OUT)

    patch_out = pl.pallas_call(
        _patch_body,
        grid=(B * P // _MT, H // _HT),
        in_specs=[
            pl.BlockSpec((_MT, D), lambda m, h: (m, 0)),
            pl.BlockSpec((_HT, D), lambda m, h: (h, 0)),
            pl.BlockSpec((1, _HT), lambda m, h: (0, h)),
            pl.BlockSpec((OUT, _HT), lambda m, h: (0, h)),
            pl.BlockSpec((1, OUT), lambda m, h: (0, 0)),
        ],
        out_specs=pl.BlockSpec((_MT, OUT), lambda m, h: (m, 0)),
        out_shape=jax.ShapeDtypeStruct((B * P, OUT), jnp.float32),
    )(patch, moe0_W1, b1r, moe0_W2, b2r)

    h_all = pl.pallas_call(
        _stage1_body,
        grid=(5,),
        in_specs=[
            pl.BlockSpec((NCLS * B, D), lambda a: (0, 0)),
            pl.BlockSpec((1, H, D), lambda a: (a, 0, 0)),
            pl.BlockSpec((1, 1, H), lambda a: (a, 0, 0)),
        ],
        out_specs=pl.BlockSpec((1, NCLS * B, H), lambda a: (a, 0, 0)),
        out_shape=jax.ShapeDtypeStruct((5, NCLS * B, H), jnp.bfloat16),
    )(tflat, atom1_W, a1b)

    pair_out = pl.pallas_call(
        _stage2_body,
        grid=(12,),
        in_specs=[
            pl.BlockSpec((1, B, D), lambda p: (_pi(p), 0, 0)),
            pl.BlockSpec((1, 2, D), lambda p: (_pi(p), 0, 0)),
            pl.BlockSpec((1, B, H), lambda p: (_pa(p), _pi(p), 0)),
            pl.BlockSpec((1, OUT, H), lambda p: (_pb(p), 0, 0)),
            pl.BlockSpec((1, 1, OUT), lambda p: (_pb(p), 0, 0)),
        ],
        out_specs=pl.BlockSpec((1, B, OUT), lambda p: (p, 0, 0)),
        out_shape=jax.ShapeDtypeStruct((12, B, OUT), jnp.float32),
    )(toks, gate_W, h_all, atom2_W, a2b)

    cls_out = pl.pallas_call(
        _combine_body,
        grid=(1,),
        in_specs=[pl.BlockSpec((12, B, OUT), lambda _: (0, 0, 0))],
        out_specs=pl.BlockSpec((NCLS, B, OUT), lambda _: (0, 0, 0)),
        out_shape=jax.ShapeDtypeStruct((NCLS, B, OUT), jnp.float32),
    )(pair_out)

    y = jnp.concatenate(
        [cls_out.transpose(1, 0, 2), patch_out.reshape(B, P, OUT)], axis=1)
    return y


# speedup vs baseline: 2.5762x; 1.2240x over previous
import jax
import jax.numpy as jnp
from jax.experimental import pallas as pl
from jax.experimental.pallas import tpu as pltpu

B = 32
NCLS = 6
P = 196
D = 768
H = 3072
OUT = 768
ROWS = NCLS + P

_PAIRS = [
    (3, 0, 0), (4, 0, 1),
    (3, 1, 2), (4, 1, 3),
    (3, 2, 4), (4, 2, 5),
    (0, 3, 0), (1, 3, 2), (2, 3, 4),
    (0, 4, 1), (1, 4, 3), (2, 4, 5),
]
_TOKEN_PAIRS = [(0, 6), (1, 9), (2, 7), (3, 10), (4, 8), (5, 11)]

_BT = 2


def _bf(v):
    return v.astype(jnp.bfloat16)


def _gelu(v):
    return 0.5 * v * (1.0 + jax.lax.erf(v * 0.7071067811865476))


def _dot_t(a, b):
    return jax.lax.dot_general(
        _bf(a), _bf(b), (((1,), (1,)), ((), ())),
        preferred_element_type=jnp.float32)


def _dot_t_bf(a, b):
    return jax.lax.dot_general(
        a, b, (((1,), (1,)), ((), ())), preferred_element_type=jnp.float32)


def _pi(p):
    return jnp.where(p < 6, p, jnp.where(p < 9, 2 * (p - 6), 2 * (p - 9) + 1))


def _pa(p):
    return jnp.where(p < 6, 3 + p % 2, (p - 6) % 3)


def _pb(p):
    return jnp.where(p < 6, p // 2, jnp.where(p < 9, 3, 4))


def _stage1_body(t_ref, w_ref, b_ref, h_ref):
    z = _dot_t(t_ref[...], w_ref[0]) + b_ref[0]
    h_ref[0] = _bf(_gelu(z))


def _stage2_body(tok_ref, gw_ref, h_ref, w_ref, b_ref, o_ref):
    o = _dot_t_bf(h_ref[0], _bf(w_ref[0])) + b_ref[0]
    g = _dot_t(tok_ref[0], gw_ref[0])
    m = jnp.max(g, axis=-1, keepdims=True)
    e = jnp.exp(g - m)
    s = e / jnp.sum(e, axis=-1, keepdims=True)
    gk = jnp.min(s, axis=-1, keepdims=True)
    w = (s - gk > 0).astype(jnp.float32)
    p = pl.program_id(0)
    wj = jnp.where(p < 6, w[:, 1:2], w[:, 0:1])
    o_ref[0] = o * wj


def _patch_body(x_ref, w1_ref, b1_ref, w2_ref, b2_ref, pair_ref, o_ref,
                w1b_ref, w2b_ref):
    t = pl.program_id(0)

    @pl.when(t == 0)
    def _():
        w1b_ref[...] = _bf(w1_ref[...])
        w2b_ref[...] = _bf(w2_ref[...])

    for q in range(_BT):
        xb = _bf(x_ref[q])
        z = _dot_t_bf(xb, w1b_ref[...]) + b1_ref[...]
        z = _bf(_gelu(z))
        o = _dot_t_bf(z, w2b_ref[...]) + b2_ref[...]
        o_ref[q] = o
        qg = t * _BT + q
        for i, (p0, p1) in enumerate(_TOKEN_PAIRS):
            o_ref[q, i:i + 1, :] = (pair_ref[p0, pl.ds(qg, 1), :] +
                                    pair_ref[p1, pl.ds(qg, 1), :])


def kernel(x, mids, gate_W, moe0_W1, moe0_b1, moe0_W2, moe0_b2,
           atom1_W, atom1_b, atom2_W, atom2_b):
    del mids
    toks = x[:, :NCLS, :].transpose(1, 0, 2)
    tflat = toks.reshape(NCLS * B, D)
    b1r = moe0_b1.reshape(1, H)
    b2r = moe0_b2.reshape(1, OUT)
    a1b = atom1_b.reshape(5, 1, H)
    a2b = atom2_b.reshape(5, 1, OUT)

    h_all = pl.pallas_call(
        _stage1_body,
        grid=(5,),
        in_specs=[
            pl.BlockSpec((NCLS * B, D), lambda a: (0, 0)),
            pl.BlockSpec((1, H, D), lambda a: (a, 0, 0)),
            pl.BlockSpec((1, 1, H), lambda a: (a, 0, 0)),
        ],
        out_specs=pl.BlockSpec((1, NCLS * B, H), lambda a: (a, 0, 0)),
        out_shape=jax.ShapeDtypeStruct((5, NCLS * B, H), jnp.bfloat16),
    )(tflat, atom1_W, a1b)

    pair_out = pl.pallas_call(
        _stage2_body,
        grid=(12,),
        in_specs=[
            pl.BlockSpec((1, B, D), lambda p: (_pi(p), 0, 0)),
            pl.BlockSpec((1, 2, D), lambda p: (_pi(p), 0, 0)),
            pl.BlockSpec((1, B, H), lambda p: (_pa(p), _pi(p), 0)),
            pl.BlockSpec((1, OUT, H), lambda p: (_pb(p), 0, 0)),
            pl.BlockSpec((1, 1, OUT), lambda p: (_pb(p), 0, 0)),
        ],
        out_specs=pl.BlockSpec((1, B, OUT), lambda p: (p, 0, 0)),
        out_shape=jax.ShapeDtypeStruct((12, B, OUT), jnp.float32),
    )(toks, gate_W, h_all, atom2_W, a2b)

    y = pl.pallas_call(
        _patch_body,
        grid=(B // _BT,),
        in_specs=[
            pl.BlockSpec((_BT, ROWS, D), lambda t: (t, 0, 0)),
            pl.BlockSpec((H, D), lambda t: (0, 0)),
            pl.BlockSpec((1, H), lambda t: (0, 0)),
            pl.BlockSpec((OUT, H), lambda t: (0, 0)),
            pl.BlockSpec((1, OUT), lambda t: (0, 0)),
            pl.BlockSpec((12, B, OUT), lambda t: (0, 0, 0)),
        ],
        out_specs=pl.BlockSpec((_BT, ROWS, OUT), lambda t: (t, 0, 0)),
        out_shape=jax.ShapeDtypeStruct((B, ROWS, OUT), jnp.float32),
        scratch_shapes=[
            pltpu.VMEM((H, D), jnp.bfloat16),
            pltpu.VMEM((OUT, H), jnp.bfloat16),
        ],
    )(x, moe0_W1, b1r, moe0_W2, b2r, pair_out)

    return y
